# baseline (speedup 1.0000x reference)
import functools

import jax
import jax.numpy as jnp
import numpy as np
from jax.experimental import pallas as pl
from jax.experimental.pallas import tpu as pltpu

E = 8
TOPK = 2
OUT_LEN = 128
B = 64
L = 4096
LF = L // 2 + 1


def _gating_conv_kernel(fp_ref, wg1_ref, gb1_ref, wg2_ref, gb2_ref, out_ref):
    fp = fp_ref[0]
    xg = jnp.concatenate([fp[1 + k:2050 + k, :] for k in range(5)], axis=1)
    h = jnp.maximum(
        jnp.dot(xg, wg1_ref[:], preferred_element_type=jnp.float32)
        + gb1_ref[0], 0.0)
    hp = jnp.concatenate(
        [jnp.zeros((2, 32), jnp.float32), h,
         jnp.zeros((5, 32), jnp.float32)], axis=0)
    patch = jnp.concatenate([hp[k:k + LF] for k in range(5)], axis=1)
    h2 = jnp.maximum(
        jnp.dot(patch, wg2_ref[:], preferred_element_type=jnp.float32)
        + gb2_ref[0], 0.0)
    out_ref[0, 0] = jnp.sum(h2, axis=0) * (1.0 / LF)


def _router_kernel(pooled_ref, mw1_ref, mb1_ref, mw2_ref, mb2_ref,
                   idx_ref, tw_ref, aux_ref):
    pooled = pooled_ref[:]
    h = jnp.maximum(
        jnp.dot(pooled, mw1_ref[:], preferred_element_type=jnp.float32)
        + mb1_ref[0], 0.0)
    logits = (jnp.dot(h, mw2_ref[:], preferred_element_type=jnp.float32)
              + mb2_ref[0])
    m = jnp.max(logits, axis=1, keepdims=True)
    ex = jnp.exp(logits - m)
    rw = ex / jnp.sum(ex, axis=1, keepdims=True)
    f_i = jnp.sum(rw, axis=0) * (1.0 / B)
    p_i = jnp.sum(logits, axis=0) * (1.0 / B)
    aux_ref[:] = (0.01 * E * jnp.sum(f_i * p_i)).reshape(1, 1)
    col = jax.lax.broadcasted_iota(jnp.int32, (B, E), 1)
    m1 = jnp.max(rw, axis=1, keepdims=True)
    i1 = jnp.min(jnp.where(rw == m1, col, E), axis=1, keepdims=True)
    masked = jnp.where(col == i1, -1.0, rw)
    m2 = jnp.max(masked, axis=1, keepdims=True)
    i2 = jnp.min(jnp.where(masked == m2, col, E), axis=1, keepdims=True)
    s = m1 + m2
    idx_ref[:] = jnp.concatenate([i1, i2], axis=1)
    tw_ref[:] = jnp.concatenate([m1 / s, m2 / s], axis=1)


def _expert_one(x1ph, w1, b1, w2c, b2, w3c, b3):
    h1 = jnp.maximum(
        jnp.dot(x1ph, w1, preferred_element_type=jnp.float32) + b1, 0.0)
    h1 = h1.astype(jnp.bfloat16)
    z32 = jnp.zeros((1, 32), jnp.bfloat16)
    p1 = [jnp.concatenate([z32, h1[128 * r:128 * (r + 1)], z32], axis=0)
          for r in range(8)]
    h2s = []
    for s in range(4):
        t = [2 * s + k - 3 for k in range(8)]
        patch = jnp.concatenate(
            [p1[tk % 8][1 + tk // 8:129 + tk // 8] for tk in t], axis=1)
        h2s.append(jnp.maximum(
            jnp.dot(patch, w2c, preferred_element_type=jnp.float32)
            + b2, 0.0).astype(jnp.bfloat16))
    z64 = jnp.zeros((1, 64), jnp.bfloat16)
    p2 = [jnp.concatenate([z64, h2s[s], z64], axis=0) for s in range(4)]
    out_ph = []
    for p in range(2):
        u = [2 * p + k - 3 for k in range(8)]
        patch = jnp.concatenate(
            [p2[uk % 4][1 + uk // 4:129 + uk // 4] for uk in u], axis=1)
        out_ph.append(jnp.maximum(
            jnp.dot(patch, w3c, preferred_element_type=jnp.float32)
            + b3, 0.0))
    return out_ph


def _expert_kernel(idx_ref, x1_ref, tw_ref,
                   wa1_ref, wa2_ref, wa3_ref, ba1_ref, ba2_ref, ba3_ref,
                   wb1_ref, wb2_ref, wb3_ref, bb1_ref, bb2_ref, bb3_ref,
                   out_ref):
    del idx_ref
    t = pl.program_id(0)
    fp = x1_ref[0]
    rows = []
    for r in range(8):
        ts = [2 * r + k for k in range(8)]
        rows.append(jnp.concatenate(
            [fp[tk % 16, tk // 16:tk // 16 + 128, :] for tk in ts], axis=1))
    x1ph = jnp.concatenate(rows, axis=0)
    fae, fao = _expert_one(x1ph, wa1_ref[0], ba1_ref[0, 0], wa2_ref[0],
                           ba2_ref[0, 0], wa3_ref[0], ba3_ref[0, 0])
    fbe, fbo = _expert_one(x1ph, wb1_ref[0], bb1_ref[0, 0], wb2_ref[0],
                           bb2_ref[0, 0], wb3_ref[0], bb3_ref[0, 0])
    row = tw_ref[pl.ds(t, 1), :]
    wa = row[:, 0:1]
    wb = row[:, 1:2]
    mx = jnp.maximum(wa * fae + wb * fbe, wa * fao + wb * fbo)
    out_ref[0] = mx


@jax.jit
def kernel(x, ew1, eb1, ew2, eb2, ew3, eb3, gw1, gb1, gw2, gb2,
           mw1, mb1, mw2, mb2):
    n = jnp.arange(L, dtype=jnp.float32)
    window = 0.5 * (1.0 - jnp.cos(2.0 * jnp.pi * n / L))
    f = jnp.fft.rfft(x * window[None, :], norm='ortho')
    feat = jnp.stack([jnp.real(f), jnp.imag(f)], axis=2).astype(jnp.float32)

    featp = jnp.pad(feat, ((0, 0), (3, 4), (0, 0)))
    fp16 = jnp.pad(feat, ((0, 0), (3, 2112 - 3 - LF), (0, 0)))
    fp16 = fp16.reshape(B, 132, 16, 2).transpose(0, 2, 1, 3)
    fp16 = fp16.astype(jnp.bfloat16)
    wg1 = gw1.transpose(2, 1, 0).reshape(10, 32)
    wg2 = gw2.transpose(2, 1, 0).reshape(160, 64)

    pooled = pl.pallas_call(
        _gating_conv_kernel,
        grid=(B,),
        in_specs=[
            pl.BlockSpec((1, 2056, 2), lambda i: (i, 0, 0)),
            pl.BlockSpec((10, 32), lambda i: (0, 0)),
            pl.BlockSpec((1, 32), lambda i: (0, 0)),
            pl.BlockSpec((160, 64), lambda i: (0, 0)),
            pl.BlockSpec((1, 64), lambda i: (0, 0)),
        ],
        out_specs=pl.BlockSpec((1, 1, 64), lambda i: (i, 0, 0)),
        out_shape=jax.ShapeDtypeStruct((B, 1, 64), jnp.float32),
        compiler_params=pltpu.CompilerParams(
            dimension_semantics=("parallel",)),
    )(featp, wg1, gb1.reshape(1, 32), wg2, gb2.reshape(1, 64))
    pooled = pooled.reshape(B, 64)

    idx, tw, aux = pl.pallas_call(
        _router_kernel,
        out_shape=(
            jax.ShapeDtypeStruct((B, TOPK), jnp.int32),
            jax.ShapeDtypeStruct((B, TOPK), jnp.float32),
            jax.ShapeDtypeStruct((1, 1), jnp.float32),
        ),
    )(pooled, mw1.T, mb1.reshape(1, 128), mw2.T, mb2.reshape(1, 8))

    flat_idx = idx.reshape(-1)

    w1f = ew1.transpose(0, 3, 2, 1).reshape(E, 16, 32).astype(jnp.bfloat16)
    w2c = ew2.transpose(0, 3, 2, 1).reshape(E, 256, 64).astype(jnp.bfloat16)
    w3c = ew3.transpose(0, 3, 2, 1).reshape(E, 512, 128).astype(jnp.bfloat16)

    def amap(nd):
        def f(i, idx_s):
            return (idx_s[2 * i],) + (0,) * nd
        return f

    def bmap(nd):
        def f(i, idx_s):
            return (idx_s[2 * i + 1],) + (0,) * nd
        return f

    def wspecs(mapper):
        return [
            pl.BlockSpec((1, 16, 32), mapper(2)),
            pl.BlockSpec((1, 256, 64), mapper(2)),
            pl.BlockSpec((1, 512, 128), mapper(2)),
            pl.BlockSpec((1, 1, 32), mapper(2)),
            pl.BlockSpec((1, 1, 64), mapper(2)),
            pl.BlockSpec((1, 1, 128), mapper(2)),
        ]

    resized = pl.pallas_call(
        _expert_kernel,
        grid_spec=pltpu.PrefetchScalarGridSpec(
            num_scalar_prefetch=1,
            grid=(B,),
            in_specs=[
                pl.BlockSpec((1, 16, 132, 2), lambda i, s: (i, 0, 0, 0)),
                pl.BlockSpec((B, TOPK), lambda i, s: (0, 0)),
            ] + wspecs(amap) + wspecs(bmap),
            out_specs=pl.BlockSpec((1, 128, 128), lambda i, s: (i, 0, 0)),
        ),
        out_shape=jax.ShapeDtypeStruct((B, 128, OUT_LEN), jnp.float32),
        compiler_params=pltpu.CompilerParams(
            dimension_semantics=("arbitrary",)),
    )(flat_idx, fp16, tw,
      w1f, w2c, w3c, eb1[:, None], eb2[:, None], eb3[:, None],
      w1f, w2c, w3c, eb1[:, None], eb2[:, None], eb3[:, None])

    return (resized.transpose(0, 2, 1), aux[0, 0])


# speedup vs baseline: 1.0997x; 1.0997x over previous
import functools

import jax
import jax.numpy as jnp
import numpy as np
from jax.experimental import pallas as pl
from jax.experimental.pallas import tpu as pltpu

E = 8
TOPK = 2
OUT_LEN = 128
B = 64
L = 4096
LF = L // 2 + 1


GT = 4


def _gating_conv_kernel(fp_ref, wg1_ref, gb1_ref, wg2_ref, gb2_ref, out_ref):
    xg = jnp.concatenate(
        [jnp.concatenate([fp_ref[j, 1 + k:2050 + k, :] for k in range(5)],
                         axis=1) for j in range(GT)], axis=0)
    h = jnp.maximum(
        jnp.dot(xg, wg1_ref[:], preferred_element_type=jnp.float32)
        + gb1_ref[0], 0.0)
    z2 = jnp.zeros((2, 32), jnp.float32)
    z5 = jnp.zeros((5, 32), jnp.float32)
    patches = []
    for j in range(GT):
        hp = jnp.concatenate([z2, h[LF * j:LF * (j + 1)], z5], axis=0)
        patches.append(
            jnp.concatenate([hp[k:k + LF] for k in range(5)], axis=1))
    patch = jnp.concatenate(patches, axis=0)
    h2 = jnp.maximum(
        jnp.dot(patch, wg2_ref[:], preferred_element_type=jnp.float32)
        + gb2_ref[0], 0.0)
    for j in range(GT):
        out_ref[j, 0] = jnp.sum(h2[LF * j:LF * (j + 1)], axis=0) * (1.0 / LF)


def _router_kernel(pooled_ref, mw1_ref, mb1_ref, mw2_ref, mb2_ref,
                   idx_ref, tw_ref, aux_ref):
    pooled = pooled_ref[:]
    h = jnp.maximum(
        jnp.dot(pooled, mw1_ref[:], preferred_element_type=jnp.float32)
        + mb1_ref[0], 0.0)
    logits = (jnp.dot(h, mw2_ref[:], preferred_element_type=jnp.float32)
              + mb2_ref[0])
    m = jnp.max(logits, axis=1, keepdims=True)
    ex = jnp.exp(logits - m)
    rw = ex / jnp.sum(ex, axis=1, keepdims=True)
    f_i = jnp.sum(rw, axis=0) * (1.0 / B)
    p_i = jnp.sum(logits, axis=0) * (1.0 / B)
    aux_ref[:] = (0.01 * E * jnp.sum(f_i * p_i)).reshape(1, 1)
    col = jax.lax.broadcasted_iota(jnp.int32, (B, E), 1)
    m1 = jnp.max(rw, axis=1, keepdims=True)
    i1 = jnp.min(jnp.where(rw == m1, col, E), axis=1, keepdims=True)
    masked = jnp.where(col == i1, -1.0, rw)
    m2 = jnp.max(masked, axis=1, keepdims=True)
    i2 = jnp.min(jnp.where(masked == m2, col, E), axis=1, keepdims=True)
    s = m1 + m2
    idx_ref[:] = jnp.concatenate([i1, i2], axis=1)
    tw_ref[:] = jnp.concatenate([m1 / s, m2 / s], axis=1)


def _expert_one(x1ph, w1, b1, w2c, b2, w3c, b3):
    h1 = jnp.maximum(
        jnp.dot(x1ph, w1, preferred_element_type=jnp.float32) + b1, 0.0)
    h1 = h1.astype(jnp.bfloat16)
    z32 = jnp.zeros((1, 32), jnp.bfloat16)
    p1 = [jnp.concatenate([z32, h1[128 * r:128 * (r + 1)], z32], axis=0)
          for r in range(8)]
    h2s = []
    for s in range(4):
        t = [2 * s + k - 3 for k in range(8)]
        patch = jnp.concatenate(
            [p1[tk % 8][1 + tk // 8:129 + tk // 8] for tk in t], axis=1)
        h2s.append(jnp.maximum(
            jnp.dot(patch, w2c, preferred_element_type=jnp.float32)
            + b2, 0.0).astype(jnp.bfloat16))
    z64 = jnp.zeros((1, 64), jnp.bfloat16)
    p2 = [jnp.concatenate([z64, h2s[s], z64], axis=0) for s in range(4)]
    out_ph = []
    for p in range(2):
        u = [2 * p + k - 3 for k in range(8)]
        patch = jnp.concatenate(
            [p2[uk % 4][1 + uk // 4:129 + uk // 4] for uk in u], axis=1)
        out_ph.append(jnp.maximum(
            jnp.dot(patch, w3c, preferred_element_type=jnp.float32)
            + b3, 0.0))
    return out_ph


def _expert_kernel(idx_ref, fp_ref, tw_ref, *refs):
    del idx_ref
    g = pl.program_id(0)
    out_ref = refs[-1]
    for j in range(2):
        (wa1, wa2, wa3, ba1, ba2, ba3,
         wb1, wb2, wb3, bb1, bb2, bb3) = refs[12 * j:12 * (j + 1)]
        fp = fp_ref[j]
        rows = []
        for r in range(8):
            ts = [2 * r + k for k in range(8)]
            rows.append(jnp.concatenate(
                [fp[tk % 16, tk // 16:tk // 16 + 128, :] for tk in ts],
                axis=1))
        x1ph = jnp.concatenate(rows, axis=0)
        fae, fao = _expert_one(x1ph, wa1[0], ba1[0, 0], wa2[0],
                               ba2[0, 0], wa3[0], ba3[0, 0])
        fbe, fbo = _expert_one(x1ph, wb1[0], bb1[0, 0], wb2[0],
                               bb2[0, 0], wb3[0], bb3[0, 0])
        row = tw_ref[pl.ds(2 * g + j, 1), :]
        wa = row[:, 0:1]
        wb = row[:, 1:2]
        mx = jnp.maximum(wa * fae + wb * fbe, wa * fao + wb * fbo)
        out_ref[j] = mx.T


@jax.jit
def kernel(x, ew1, eb1, ew2, eb2, ew3, eb3, gw1, gb1, gw2, gb2,
           mw1, mb1, mw2, mb2):
    n = jnp.arange(L, dtype=jnp.float32)
    window = 0.5 * (1.0 - jnp.cos(2.0 * jnp.pi * n / L))
    f = jnp.fft.rfft(x * window[None, :], norm='ortho')
    feat = jnp.stack([jnp.real(f), jnp.imag(f)], axis=2).astype(jnp.float32)

    featp = jnp.pad(feat, ((0, 0), (3, 4), (0, 0)))
    fp16 = jnp.pad(feat, ((0, 0), (3, 2112 - 3 - LF), (0, 0)))
    fp16 = fp16.reshape(B, 132, 16, 2).transpose(0, 2, 1, 3)
    fp16 = fp16.astype(jnp.bfloat16)
    wg1 = gw1.transpose(2, 1, 0).reshape(10, 32)
    wg2 = gw2.transpose(2, 1, 0).reshape(160, 64)

    pooled = pl.pallas_call(
        _gating_conv_kernel,
        grid=(B // GT,),
        in_specs=[
            pl.BlockSpec((GT, 2056, 2), lambda i: (i, 0, 0)),
            pl.BlockSpec((10, 32), lambda i: (0, 0)),
            pl.BlockSpec((1, 32), lambda i: (0, 0)),
            pl.BlockSpec((160, 64), lambda i: (0, 0)),
            pl.BlockSpec((1, 64), lambda i: (0, 0)),
        ],
        out_specs=pl.BlockSpec((GT, 1, 64), lambda i: (i, 0, 0)),
        out_shape=jax.ShapeDtypeStruct((B, 1, 64), jnp.float32),
        compiler_params=pltpu.CompilerParams(
            dimension_semantics=("parallel",)),
    )(featp, wg1, gb1.reshape(1, 32), wg2, gb2.reshape(1, 64))
    pooled = pooled.reshape(B, 64)

    idx, tw, aux = pl.pallas_call(
        _router_kernel,
        out_shape=(
            jax.ShapeDtypeStruct((B, TOPK), jnp.int32),
            jax.ShapeDtypeStruct((B, TOPK), jnp.float32),
            jax.ShapeDtypeStruct((1, 1), jnp.float32),
        ),
    )(pooled, mw1.T, mb1.reshape(1, 128), mw2.T, mb2.reshape(1, 8))

    flat_idx = idx.reshape(-1)

    w1f = ew1.transpose(0, 3, 2, 1).reshape(E, 16, 32).astype(jnp.bfloat16)
    w2c = ew2.transpose(0, 3, 2, 1).reshape(E, 256, 64).astype(jnp.bfloat16)
    w3c = ew3.transpose(0, 3, 2, 1).reshape(E, 512, 128).astype(jnp.bfloat16)

    def smap(o, nd):
        def f(i, idx_s):
            return (idx_s[4 * i + o],) + (0,) * nd
        return f

    def wspecs(o):
        return [
            pl.BlockSpec((1, 16, 32), smap(o, 2)),
            pl.BlockSpec((1, 256, 64), smap(o, 2)),
            pl.BlockSpec((1, 512, 128), smap(o, 2)),
            pl.BlockSpec((1, 1, 32), smap(o, 2)),
            pl.BlockSpec((1, 1, 64), smap(o, 2)),
            pl.BlockSpec((1, 1, 128), smap(o, 2)),
        ]

    def wargs():
        return (w1f, w2c, w3c, eb1[:, None], eb2[:, None], eb3[:, None])

    resized = pl.pallas_call(
        _expert_kernel,
        grid_spec=pltpu.PrefetchScalarGridSpec(
            num_scalar_prefetch=1,
            grid=(B // 2,),
            in_specs=[
                pl.BlockSpec((2, 16, 132, 2), lambda i, s: (i, 0, 0, 0)),
                pl.BlockSpec((B, TOPK), lambda i, s: (0, 0)),
            ] + (wspecs(0)[:3] + wspecs(0)[3:] + wspecs(1))
              + (wspecs(2) + wspecs(3)),
            out_specs=pl.BlockSpec((2, 128, 128), lambda i, s: (i, 0, 0)),
        ),
        out_shape=jax.ShapeDtypeStruct((B, 128, OUT_LEN), jnp.float32),
        compiler_params=pltpu.CompilerParams(
            dimension_semantics=("arbitrary",)),
    )(flat_idx, fp16, tw,
      *wargs(), *wargs(), *wargs(), *wargs())

    return (resized, aux[0, 0])


# speedup vs baseline: 1.1158x; 1.0146x over previous
import functools

import jax
import jax.numpy as jnp
import numpy as np
from jax.experimental import pallas as pl
from jax.experimental.pallas import tpu as pltpu

E = 8
TOPK = 2
OUT_LEN = 128
B = 64
L = 4096
LF = L // 2 + 1


GT = 8


def _gating_conv_kernel(fp_ref, wg1_ref, gb1_ref, wg2_ref, gb2_ref, out_ref):
    xg = jnp.concatenate(
        [jnp.concatenate([fp_ref[j, 1 + k:2050 + k, :] for k in range(5)],
                         axis=1) for j in range(GT)], axis=0)
    h = jnp.maximum(
        jnp.dot(xg, wg1_ref[:], preferred_element_type=jnp.float32)
        + gb1_ref[0], 0.0)
    z2 = jnp.zeros((2, 32), jnp.float32)
    z5 = jnp.zeros((5, 32), jnp.float32)
    patches = []
    for j in range(GT):
        hp = jnp.concatenate([z2, h[LF * j:LF * (j + 1)], z5], axis=0)
        patches.append(
            jnp.concatenate([hp[k:k + LF] for k in range(5)], axis=1))
    patch = jnp.concatenate(patches, axis=0)
    h2 = jnp.maximum(
        jnp.dot(patch, wg2_ref[:], preferred_element_type=jnp.float32)
        + gb2_ref[0], 0.0)
    for j in range(GT):
        out_ref[j, 0] = jnp.sum(h2[LF * j:LF * (j + 1)], axis=0) * (1.0 / LF)


def _router_kernel(pooled_ref, mw1_ref, mb1_ref, mw2_ref, mb2_ref,
                   idx_ref, tw_ref, aux_ref):
    pooled = pooled_ref[:]
    h = jnp.maximum(
        jnp.dot(pooled, mw1_ref[:], preferred_element_type=jnp.float32)
        + mb1_ref[0], 0.0)
    logits = (jnp.dot(h, mw2_ref[:], preferred_element_type=jnp.float32)
              + mb2_ref[0])
    m = jnp.max(logits, axis=1, keepdims=True)
    ex = jnp.exp(logits - m)
    rw = ex / jnp.sum(ex, axis=1, keepdims=True)
    f_i = jnp.sum(rw, axis=0) * (1.0 / B)
    p_i = jnp.sum(logits, axis=0) * (1.0 / B)
    aux_ref[:] = (0.01 * E * jnp.sum(f_i * p_i)).reshape(1, 1)
    col = jax.lax.broadcasted_iota(jnp.int32, (B, E), 1)
    m1 = jnp.max(rw, axis=1, keepdims=True)
    i1 = jnp.min(jnp.where(rw == m1, col, E), axis=1, keepdims=True)
    masked = jnp.where(col == i1, -1.0, rw)
    m2 = jnp.max(masked, axis=1, keepdims=True)
    i2 = jnp.min(jnp.where(masked == m2, col, E), axis=1, keepdims=True)
    s = m1 + m2
    idx_ref[:] = jnp.concatenate([i1, i2], axis=1)
    tw_ref[:] = jnp.concatenate([m1 / s, m2 / s], axis=1)


def _expert_one(x1ph, w1, b1, w2c, b2, w3c, b3):
    h1 = jnp.maximum(
        jnp.dot(x1ph, w1, preferred_element_type=jnp.float32) + b1, 0.0)
    h1 = h1.astype(jnp.bfloat16)
    z32 = jnp.zeros((1, 32), jnp.bfloat16)
    p1 = [jnp.concatenate([z32, h1[128 * r:128 * (r + 1)], z32], axis=0)
          for r in range(8)]
    h2s = []
    for s in range(4):
        t = [2 * s + k - 3 for k in range(8)]
        patch = jnp.concatenate(
            [p1[tk % 8][1 + tk // 8:129 + tk // 8] for tk in t], axis=1)
        h2s.append(jnp.maximum(
            jnp.dot(patch, w2c, preferred_element_type=jnp.float32)
            + b2, 0.0).astype(jnp.bfloat16))
    z64 = jnp.zeros((1, 64), jnp.bfloat16)
    p2 = [jnp.concatenate([z64, h2s[s], z64], axis=0) for s in range(4)]
    out_ph = []
    for p in range(2):
        u = [2 * p + k - 3 for k in range(8)]
        patch = jnp.concatenate(
            [p2[uk % 4][1 + uk // 4:129 + uk // 4] for uk in u], axis=1)
        out_ph.append(jnp.maximum(
            jnp.dot(patch, w3c, preferred_element_type=jnp.float32)
            + b3, 0.0))
    return out_ph


TPE = 4


def _expert_kernel(idx_ref, fp_ref, tw_ref, *refs):
    del idx_ref
    g = pl.program_id(0)
    out_ref = refs[-1]
    for j in range(TPE):
        (wa1, wa2, wa3, ba1, ba2, ba3,
         wb1, wb2, wb3, bb1, bb2, bb3) = refs[12 * j:12 * (j + 1)]
        fp = fp_ref[j]
        rows = []
        for r in range(8):
            ts = [2 * r + k for k in range(8)]
            rows.append(jnp.concatenate(
                [fp[tk % 16, tk // 16:tk // 16 + 128, :] for tk in ts],
                axis=1))
        x1ph = jnp.concatenate(rows, axis=0)
        fae, fao = _expert_one(x1ph, wa1[0], ba1[0, 0], wa2[0],
                               ba2[0, 0], wa3[0], ba3[0, 0])
        fbe, fbo = _expert_one(x1ph, wb1[0], bb1[0, 0], wb2[0],
                               bb2[0, 0], wb3[0], bb3[0, 0])
        row = tw_ref[pl.ds(TPE * g + j, 1), :]
        wa = row[:, 0:1]
        wb = row[:, 1:2]
        mx = jnp.maximum(wa * fae + wb * fbe, wa * fao + wb * fbo)
        out_ref[j] = mx.T


@jax.jit
def kernel(x, ew1, eb1, ew2, eb2, ew3, eb3, gw1, gb1, gw2, gb2,
           mw1, mb1, mw2, mb2):
    n = jnp.arange(L, dtype=jnp.float32)
    window = 0.5 * (1.0 - jnp.cos(2.0 * jnp.pi * n / L))
    f = jnp.fft.rfft(x * window[None, :], norm='ortho')
    feat = jnp.stack([jnp.real(f), jnp.imag(f)], axis=2).astype(jnp.float32)

    featp = jnp.pad(feat, ((0, 0), (3, 4), (0, 0)))
    fp16 = jnp.pad(feat, ((0, 0), (3, 2112 - 3 - LF), (0, 0)))
    fp16 = fp16.reshape(B, 132, 16, 2).transpose(0, 2, 1, 3)
    fp16 = fp16.astype(jnp.bfloat16)
    wg1 = gw1.transpose(2, 1, 0).reshape(10, 32)
    wg2 = gw2.transpose(2, 1, 0).reshape(160, 64)

    pooled = pl.pallas_call(
        _gating_conv_kernel,
        grid=(B // GT,),
        in_specs=[
            pl.BlockSpec((GT, 2056, 2), lambda i: (i, 0, 0)),
            pl.BlockSpec((10, 32), lambda i: (0, 0)),
            pl.BlockSpec((1, 32), lambda i: (0, 0)),
            pl.BlockSpec((160, 64), lambda i: (0, 0)),
            pl.BlockSpec((1, 64), lambda i: (0, 0)),
        ],
        out_specs=pl.BlockSpec((GT, 1, 64), lambda i: (i, 0, 0)),
        out_shape=jax.ShapeDtypeStruct((B, 1, 64), jnp.float32),
        compiler_params=pltpu.CompilerParams(
            dimension_semantics=("parallel",),
            vmem_limit_bytes=100 * 1024 * 1024),
    )(featp, wg1, gb1.reshape(1, 32), wg2, gb2.reshape(1, 64))
    pooled = pooled.reshape(B, 64)

    idx, tw, aux = pl.pallas_call(
        _router_kernel,
        out_shape=(
            jax.ShapeDtypeStruct((B, TOPK), jnp.int32),
            jax.ShapeDtypeStruct((B, TOPK), jnp.float32),
            jax.ShapeDtypeStruct((1, 1), jnp.float32),
        ),
    )(pooled, mw1.T, mb1.reshape(1, 128), mw2.T, mb2.reshape(1, 8))

    flat_idx = idx.reshape(-1)

    w1f = ew1.transpose(0, 3, 2, 1).reshape(E, 16, 32).astype(jnp.bfloat16)
    w2c = ew2.transpose(0, 3, 2, 1).reshape(E, 256, 64).astype(jnp.bfloat16)
    w3c = ew3.transpose(0, 3, 2, 1).reshape(E, 512, 128).astype(jnp.bfloat16)

    def smap(o, nd):
        def f(i, idx_s):
            return (idx_s[2 * TPE * i + o],) + (0,) * nd
        return f

    def wspecs(o):
        return [
            pl.BlockSpec((1, 16, 32), smap(o, 2)),
            pl.BlockSpec((1, 256, 64), smap(o, 2)),
            pl.BlockSpec((1, 512, 128), smap(o, 2)),
            pl.BlockSpec((1, 1, 32), smap(o, 2)),
            pl.BlockSpec((1, 1, 64), smap(o, 2)),
            pl.BlockSpec((1, 1, 128), smap(o, 2)),
        ]

    def wargs():
        return (w1f, w2c, w3c, eb1[:, None], eb2[:, None], eb3[:, None])

    resized = pl.pallas_call(
        _expert_kernel,
        grid_spec=pltpu.PrefetchScalarGridSpec(
            num_scalar_prefetch=1,
            grid=(B // TPE,),
            in_specs=[
                pl.BlockSpec((TPE, 16, 132, 2),
                             lambda i, s: (i, 0, 0, 0)),
                pl.BlockSpec((B, TOPK), lambda i, s: (0, 0)),
            ] + sum([wspecs(o) for o in range(2 * TPE)], []),
            out_specs=pl.BlockSpec((TPE, 128, 128),
                                   lambda i, s: (i, 0, 0)),
        ),
        out_shape=jax.ShapeDtypeStruct((B, 128, OUT_LEN), jnp.float32),
        compiler_params=pltpu.CompilerParams(
            dimension_semantics=("arbitrary",),
            vmem_limit_bytes=100 * 1024 * 1024),
    )(flat_idx, fp16, tw,
      *[a for _ in range(2 * TPE) for a in wargs()])

    return (resized, aux[0, 0])


# speedup vs baseline: 1.1162x; 1.0003x over previous
import functools

import jax
import jax.numpy as jnp
import numpy as np
from jax.experimental import pallas as pl
from jax.experimental.pallas import tpu as pltpu

E = 8
TOPK = 2
OUT_LEN = 128
B = 64
L = 4096
LF = L // 2 + 1


GT = 8


def _gating_conv_kernel(fp_ref, wg1_ref, gb1_ref, wg2_ref, gb2_ref, out_ref):
    xg = jnp.concatenate(
        [jnp.concatenate([fp_ref[j, 1 + k:2050 + k, :] for k in range(5)],
                         axis=1) for j in range(GT)], axis=0)
    h = jnp.maximum(
        jnp.dot(xg, wg1_ref[:], preferred_element_type=jnp.float32)
        + gb1_ref[0], 0.0)
    z2 = jnp.zeros((2, 32), jnp.float32)
    z5 = jnp.zeros((5, 32), jnp.float32)
    patches = []
    for j in range(GT):
        hp = jnp.concatenate([z2, h[LF * j:LF * (j + 1)], z5], axis=0)
        patches.append(
            jnp.concatenate([hp[k:k + LF] for k in range(5)], axis=1))
    patch = jnp.concatenate(patches, axis=0)
    h2 = jnp.maximum(
        jnp.dot(patch, wg2_ref[:], preferred_element_type=jnp.float32)
        + gb2_ref[0], 0.0)
    for j in range(GT):
        out_ref[j, 0] = jnp.sum(h2[LF * j:LF * (j + 1)], axis=0) * (1.0 / LF)


def _router_kernel(pooled_ref, mw1_ref, mb1_ref, mw2_ref, mb2_ref,
                   idx_ref, tw_ref, aux_ref):
    pooled = pooled_ref[:]
    h = jnp.maximum(
        jnp.dot(pooled, mw1_ref[:], preferred_element_type=jnp.float32)
        + mb1_ref[0], 0.0)
    logits = (jnp.dot(h, mw2_ref[:], preferred_element_type=jnp.float32)
              + mb2_ref[0])
    m = jnp.max(logits, axis=1, keepdims=True)
    ex = jnp.exp(logits - m)
    rw = ex / jnp.sum(ex, axis=1, keepdims=True)
    f_i = jnp.sum(rw, axis=0) * (1.0 / B)
    p_i = jnp.sum(logits, axis=0) * (1.0 / B)
    aux_ref[:] = (0.01 * E * jnp.sum(f_i * p_i)).reshape(1, 1)
    col = jax.lax.broadcasted_iota(jnp.int32, (B, E), 1)
    m1 = jnp.max(rw, axis=1, keepdims=True)
    i1 = jnp.min(jnp.where(rw == m1, col, E), axis=1, keepdims=True)
    masked = jnp.where(col == i1, -1.0, rw)
    m2 = jnp.max(masked, axis=1, keepdims=True)
    i2 = jnp.min(jnp.where(masked == m2, col, E), axis=1, keepdims=True)
    s = m1 + m2
    idx_ref[:] = jnp.concatenate([i1, i2], axis=1)
    tw_ref[:] = jnp.concatenate([m1 / s, m2 / s], axis=1)


def _expert_one(x1ph, w1, b1, w2c, b2, w3c, b3):
    h1 = jnp.maximum(
        jnp.dot(x1ph, w1, preferred_element_type=jnp.float32) + b1, 0.0)
    h1 = h1.astype(jnp.bfloat16)
    z32 = jnp.zeros((1, 32), jnp.bfloat16)
    p1 = [jnp.concatenate([z32, h1[128 * r:128 * (r + 1)], z32], axis=0)
          for r in range(8)]
    h2s = []
    for s in range(4):
        t = [2 * s + k - 3 for k in range(8)]
        patch = jnp.concatenate(
            [p1[tk % 8][1 + tk // 8:129 + tk // 8] for tk in t], axis=1)
        h2s.append(jnp.maximum(
            jnp.dot(patch, w2c, preferred_element_type=jnp.float32)
            + b2, 0.0).astype(jnp.bfloat16))
    z64 = jnp.zeros((1, 64), jnp.bfloat16)
    p2 = [jnp.concatenate([z64, h2s[s], z64], axis=0) for s in range(4)]
    out_ph = []
    for p in range(2):
        u = [2 * p + k - 3 for k in range(8)]
        patch = jnp.concatenate(
            [p2[uk % 4][1 + uk // 4:129 + uk // 4] for uk in u], axis=1)
        out_ph.append(jnp.maximum(
            jnp.dot(patch, w3c, preferred_element_type=jnp.float32)
            + b3, 0.0))
    return out_ph


TPE = 4


def _expert_kernel(idx_ref, fp_ref, tw_ref, *refs):
    del idx_ref
    g = pl.program_id(0)
    out_ref = refs[-1]
    for j in range(TPE):
        (wa1, wa2, wa3, ba1, ba2, ba3,
         wb1, wb2, wb3, bb1, bb2, bb3) = refs[12 * j:12 * (j + 1)]
        fp = fp_ref[j]
        rows = []
        for r in range(8):
            ts = [2 * r + k for k in range(8)]
            rows.append(jnp.concatenate(
                [fp[tk % 16, tk // 16:tk // 16 + 128, :] for tk in ts],
                axis=1))
        x1ph = jnp.concatenate(rows, axis=0)
        fae, fao = _expert_one(x1ph, wa1[0], ba1[0, 0], wa2[0],
                               ba2[0, 0], wa3[0], ba3[0, 0])
        fbe, fbo = _expert_one(x1ph, wb1[0], bb1[0, 0], wb2[0],
                               bb2[0, 0], wb3[0], bb3[0, 0])
        row = tw_ref[pl.ds(TPE * g + j, 1), :]
        wa = row[:, 0:1]
        wb = row[:, 1:2]
        mx = jnp.maximum(wa * fae + wb * fbe, wa * fao + wb * fbo)
        out_ref[j] = mx.T


@jax.jit
def kernel(x, ew1, eb1, ew2, eb2, ew3, eb3, gw1, gb1, gw2, gb2,
           mw1, mb1, mw2, mb2):
    n = jnp.arange(L, dtype=jnp.float32)
    window = 0.5 * (1.0 - jnp.cos(2.0 * jnp.pi * n / L))
    f = jnp.fft.rfft(x * window[None, :], norm='ortho')
    feat = jnp.stack([jnp.real(f), jnp.imag(f)], axis=2).astype(jnp.float32)

    featp = jnp.pad(feat, ((0, 0), (3, 4), (0, 0)))
    fp16 = jnp.pad(feat, ((0, 0), (3, 2112 - 3 - LF), (0, 0)))
    fp16 = fp16.reshape(B, 132, 16, 2).transpose(0, 2, 1, 3)
    fp16 = fp16.astype(jnp.bfloat16)
    wg1 = gw1.transpose(2, 1, 0).reshape(10, 32)
    wg2 = gw2.transpose(2, 1, 0).reshape(160, 64)

    pooled = pl.pallas_call(
        _gating_conv_kernel,
        grid=(B // GT,),
        in_specs=[
            pl.BlockSpec((GT, 2056, 2), lambda i: (i, 0, 0)),
            pl.BlockSpec((10, 32), lambda i: (0, 0)),
            pl.BlockSpec((1, 32), lambda i: (0, 0)),
            pl.BlockSpec((160, 64), lambda i: (0, 0)),
            pl.BlockSpec((1, 64), lambda i: (0, 0)),
        ],
        out_specs=pl.BlockSpec((GT, 1, 64), lambda i: (i, 0, 0)),
        out_shape=jax.ShapeDtypeStruct((B, 1, 64), jnp.float32),
        compiler_params=pltpu.CompilerParams(
            dimension_semantics=("parallel",),
            vmem_limit_bytes=100 * 1024 * 1024),
    )(featp, wg1, gb1.reshape(1, 32), wg2, gb2.reshape(1, 64))
    pooled = pooled.reshape(B, 64)

    idx, tw, aux = pl.pallas_call(
        _router_kernel,
        out_shape=(
            jax.ShapeDtypeStruct((B, TOPK), jnp.int32),
            jax.ShapeDtypeStruct((B, TOPK), jnp.float32),
            jax.ShapeDtypeStruct((1, 1), jnp.float32),
        ),
    )(pooled, mw1.T, mb1.reshape(1, 128), mw2.T, mb2.reshape(1, 8))

    flat_idx = idx.reshape(-1)

    w1f = ew1.transpose(0, 3, 2, 1).reshape(E, 16, 32).astype(jnp.bfloat16)
    w2c = ew2.transpose(0, 3, 2, 1).reshape(E, 256, 64).astype(jnp.bfloat16)
    w3c = ew3.transpose(0, 3, 2, 1).reshape(E, 512, 128).astype(jnp.bfloat16)

    def smap(o, nd):
        def f(i, idx_s):
            return (idx_s[2 * TPE * i + o],) + (0,) * nd
        return f

    def wspecs(o):
        return [
            pl.BlockSpec((1, 16, 32), smap(o, 2)),
            pl.BlockSpec((1, 256, 64), smap(o, 2)),
            pl.BlockSpec((1, 512, 128), smap(o, 2)),
            pl.BlockSpec((1, 1, 32), smap(o, 2)),
            pl.BlockSpec((1, 1, 64), smap(o, 2)),
            pl.BlockSpec((1, 1, 128), smap(o, 2)),
        ]

    def wargs():
        return (w1f, w2c, w3c, eb1[:, None], eb2[:, None], eb3[:, None])

    resized = pl.pallas_call(
        _expert_kernel,
        grid_spec=pltpu.PrefetchScalarGridSpec(
            num_scalar_prefetch=1,
            grid=(B // TPE,),
            in_specs=[
                pl.BlockSpec((TPE, 16, 132, 2),
                             lambda i, s: (i, 0, 0, 0)),
                pl.BlockSpec((B, TOPK), lambda i, s: (0, 0)),
            ] + sum([wspecs(o) for o in range(2 * TPE)], []),
            out_specs=pl.BlockSpec((TPE, 128, 128),
                                   lambda i, s: (i, 0, 0)),
        ),
        out_shape=jax.ShapeDtypeStruct((B, 128, OUT_LEN), jnp.float32),
        compiler_params=pltpu.CompilerParams(
            dimension_semantics=("parallel",),
            vmem_limit_bytes=100 * 1024 * 1024),
    )(flat_idx, fp16, tw,
      *[a for _ in range(2 * TPE) for a in wargs()])

    return (resized, aux[0, 0])
